# baseline (device time: 82326 ns/iter reference)
import jax
import jax.numpy as jnp
from jax import lax
from jax.experimental import pallas as pl
from jax.experimental.pallas import tpu as pltpu

N_DEV = 4
TAPS = 4


def kernel(x, k, Wp):
    b, s, c = x.shape
    n_out = Wp.shape[1]

    def body(x_ref, k_ref, Wp_ref, out_ref, comm_ref, send_sems, recv_sems):
        my = lax.axis_index("i")
        left = lax.rem(my + (N_DEV - 1), N_DEV)
        right = lax.rem(my + 1, N_DEV)

        barrier_sem = pltpu.get_barrier_semaphore()
        for nbr in (left, right):
            pl.semaphore_signal(
                barrier_sem, inc=1,
                device_id=(nbr,), device_id_type=pl.DeviceIdType.MESH,
            )
        pl.semaphore_wait(barrier_sem, 2)

        xv = x_ref[...]
        conv = xv * k_ref[TAPS - 1]
        for t in range(TAPS - 1):
            shift = TAPS - 1 - t
            shifted = jnp.concatenate(
                [jnp.zeros((b, shift, c), jnp.float32), xv[:, : s - shift, :]],
                axis=1,
            )
            conv = conv + shifted * k_ref[t]
        a = conv * jax.nn.sigmoid(conv)
        partial = lax.dot_general(
            a.reshape(b * s, c), Wp_ref[...],
            (((1,), (0,)), ((), ())),
            preferred_element_type=jnp.float32,
        ).reshape(b, s, n_out)

        out_ref[...] = partial
        comm_ref[0] = partial

        for h in range(N_DEV - 1):
            rdma = pltpu.make_async_remote_copy(
                src_ref=comm_ref.at[h],
                dst_ref=comm_ref.at[h + 1],
                send_sem=send_sems.at[h],
                recv_sem=recv_sems.at[h],
                device_id=(right,),
                device_id_type=pl.DeviceIdType.MESH,
            )
            rdma.start()
            rdma.wait()
            out_ref[...] += comm_ref[h + 1]

    return pl.pallas_call(
        body,
        out_shape=jax.ShapeDtypeStruct((b, s, n_out), jnp.float32),
        in_specs=[
            pl.BlockSpec(memory_space=pltpu.VMEM),
            pl.BlockSpec(memory_space=pltpu.VMEM),
            pl.BlockSpec(memory_space=pltpu.VMEM),
        ],
        out_specs=pl.BlockSpec(memory_space=pltpu.VMEM),
        scratch_shapes=[
            pltpu.VMEM((N_DEV, b, s, n_out), jnp.float32),
            pltpu.SemaphoreType.DMA((N_DEV - 1,)),
            pltpu.SemaphoreType.DMA((N_DEV - 1,)),
        ],
        compiler_params=pltpu.CompilerParams(collective_id=0),
    )(x, k, Wp)


# device time: 32832 ns/iter; 2.5075x vs baseline; 2.5075x over previous
import jax
import jax.numpy as jnp
from jax import lax
from jax.experimental import pallas as pl
from jax.experimental.pallas import tpu as pltpu

N_DEV = 4
TAPS = 4


def kernel(x, k, Wp):
    b, s, c = x.shape
    n_out = Wp.shape[1]
    R = b * s
    H = R // 2
    Q = R // 4
    E = R // 8

    def body(
        x_ref, k_ref, Wp_ref, out_ref, acc_ref,
        rA1, rA2, rA3, rA4, rB1, rB2, rB3, rB4,
        send_sems, recv_sems,
    ):
        p = lax.axis_index("i")
        q_y = p ^ 1
        q_x = 3 - p
        b_y = (p ^ (p >> 1)) & 1
        b_x = (p >> 1) & 1

        barrier_sem = pltpu.get_barrier_semaphore()
        for nbr in (q_y, q_x):
            pl.semaphore_signal(
                barrier_sem, inc=1,
                device_id=(nbr,), device_id_type=pl.DeviceIdType.MESH,
            )
        pl.semaphore_wait(barrier_sem, 2)

        xv = x_ref[...]
        conv = xv * k_ref[TAPS - 1]
        for t in range(TAPS - 1):
            shift = TAPS - 1 - t
            shifted = jnp.concatenate(
                [jnp.zeros((b, shift, c), jnp.float32), xv[:, : s - shift, :]],
                axis=1,
            )
            conv = conv + shifted * k_ref[t]
        a = conv * jax.nn.sigmoid(conv)
        acc_ref[...] = lax.dot_general(
            a.reshape(R, c), Wp_ref[...],
            (((1,), (0,)), ((), ())),
            preferred_element_type=jnp.float32,
        )

        def exchange(sem_idx, src_off, n_rows, dst_ref, partner):
            rdma = pltpu.make_async_remote_copy(
                src_ref=acc_ref.at[pl.ds(src_off, n_rows), :],
                dst_ref=dst_ref,
                send_sem=send_sems.at[sem_idx],
                recv_sem=recv_sems.at[sem_idx],
                device_id=(partner,),
                device_id_type=pl.DeviceIdType.MESH,
            )
            rdma.start()
            return rdma

        keepQ_A = b_y * Q
        sendQ_A = (1 - b_y) * Q
        keepQ_B = H + b_x * Q
        sendQ_B = H + (1 - b_x) * Q
        keepE_A = keepQ_A + b_x * E
        sendE_A = keepQ_A + (1 - b_x) * E
        keepE_B = keepQ_B + b_y * E
        sendE_B = keepQ_B + (1 - b_y) * E

        r1a = exchange(0, sendQ_A, Q, rA1, q_y)
        r1b = exchange(1, sendQ_B, Q, rB1, q_x)
        r1a.wait()
        r1b.wait()
        acc_ref[pl.ds(keepQ_A, Q), :] += rA1[...]
        acc_ref[pl.ds(keepQ_B, Q), :] += rB1[...]

        r2a = exchange(2, sendE_A, E, rA2, q_x)
        r2b = exchange(3, sendE_B, E, rB2, q_y)
        r2a.wait()
        r2b.wait()
        acc_ref[pl.ds(keepE_A, E), :] += rA2[...]
        acc_ref[pl.ds(keepE_B, E), :] += rB2[...]

        r3a = exchange(4, keepE_A, E, rA3, q_x)
        r3b = exchange(5, keepE_B, E, rB3, q_y)
        r3a.wait()
        r3b.wait()
        acc_ref[pl.ds(sendE_A, E), :] = rA3[...]
        acc_ref[pl.ds(sendE_B, E), :] = rB3[...]

        r4a = exchange(6, keepQ_A, Q, rA4, q_y)
        r4b = exchange(7, keepQ_B, Q, rB4, q_x)
        r4a.wait()
        r4b.wait()
        acc_ref[pl.ds(sendQ_A, Q), :] = rA4[...]
        acc_ref[pl.ds(sendQ_B, Q), :] = rB4[...]

        out_ref[...] = acc_ref[...].reshape(b, s, n_out)

    return pl.pallas_call(
        body,
        out_shape=jax.ShapeDtypeStruct((b, s, n_out), jnp.float32),
        in_specs=[
            pl.BlockSpec(memory_space=pltpu.VMEM),
            pl.BlockSpec(memory_space=pltpu.VMEM),
            pl.BlockSpec(memory_space=pltpu.VMEM),
        ],
        out_specs=pl.BlockSpec(memory_space=pltpu.VMEM),
        scratch_shapes=[
            pltpu.VMEM((R, n_out), jnp.float32),
            pltpu.VMEM((Q, n_out), jnp.float32),
            pltpu.VMEM((E, n_out), jnp.float32),
            pltpu.VMEM((E, n_out), jnp.float32),
            pltpu.VMEM((Q, n_out), jnp.float32),
            pltpu.VMEM((Q, n_out), jnp.float32),
            pltpu.VMEM((E, n_out), jnp.float32),
            pltpu.VMEM((E, n_out), jnp.float32),
            pltpu.VMEM((Q, n_out), jnp.float32),
            pltpu.SemaphoreType.DMA((8,)),
            pltpu.SemaphoreType.DMA((8,)),
        ],
        compiler_params=pltpu.CompilerParams(collective_id=0),
    )(x, k, Wp)
